# Initial kernel scaffold; baseline (speedup 1.0000x reference)
import jax
import jax.numpy as jnp
from jax.experimental import pallas as pl


def kernel(x, eps, w1_mu, b1_mu, w2_mu, b2_mu, w1_sigma, b1_sigma, w2_sigma, b2_sigma):
    raise NotImplementedError("write your pallas kernel here")


# speedup vs baseline: 2.0830x; 2.0830x over previous
import math

import jax
import jax.numpy as jnp
from jax.experimental import pallas as pl
from jax.experimental.pallas import tpu as pltpu

_LOG_SQRT_2PI = 0.5 * math.log(2.0 * math.pi)
_BLK = 64
_VMEM_LIMIT = 64 * 1024 * 1024


def _vshift(x, k):
    if k == 0:
        return x
    return jnp.concatenate((x[:, k:], x[:, :k]), axis=1)


def _fused_kernel(xp_ref, w1_ref, b1_ref, w2_ref, b2_ref, eps_ref,
                  z_ref, lp_ref):
    xb = xp_ref[...]
    blk = xb.shape[0]

    xcat = jnp.concatenate([_vshift(xb, a) for a in range(4)], axis=2)
    xcat = xcat.reshape(blk * 16, 256)
    y1 = jnp.dot(xcat, w1_ref[...], preferred_element_type=jnp.float32)
    m = jnp.maximum(jnp.maximum(y1[:, 0:176], y1[:, 176:352]),
                    jnp.maximum(y1[:, 352:528], y1[:, 528:704]))
    h = jnp.maximum(m + b1_ref[...], 0.0)
    h3 = h.reshape(blk, 16, 176)

    g = jnp.concatenate([_vshift(h3, ky) for ky in range(6)], axis=2)
    g = g.reshape(blk * 16, 1056)
    y2 = jnp.dot(g, w2_ref[...], preferred_element_type=jnp.float32)
    y2 = jnp.maximum(jnp.maximum(y2[:, 0:60], y2[:, 60:120]),
                     jnp.maximum(y2[:, 120:180], y2[:, 180:240]))
    y2 = (y2 + b2_ref[...]).reshape(blk, 16, 60)

    out = jnp.stack([y2[:, 0], y2[:, 2], y2[:, 4]], axis=1)
    parts = [out[:, :, o * 20:(o + 1) * 20] for o in range(3)]
    y = jnp.stack(parts, axis=2)

    mu = y[..., 0:10]
    logits = y[..., 10:20]
    e = jnp.exp(logits - jnp.max(logits, axis=-1, keepdims=True))
    sigma = e / jnp.sum(e, axis=-1, keepdims=True)
    epsv = eps_ref[...]
    z_ref[...] = mu + sigma * epsv
    lp = -0.5 * epsv * epsv - jnp.log(sigma) - _LOG_SQRT_2PI
    lp_ref[...] = jnp.sum(lp, axis=(1, 2), keepdims=True)


def _prep_w1(w1_mu, w1_sigma):
    w1 = jnp.concatenate([w1_mu[:, 0], w1_sigma[:, 0]], 0)
    shifted = []
    for d1 in range(2):
        for d2 in range(2):
            shifted.append(jnp.pad(w1, ((0, 0), (d1, 1 - d1), (d2, 1 - d2))))
    wp = jnp.stack(shifted, 0)
    t = wp.reshape(4, 16, 4, 2, 4, 2)
    jj = jnp.arange(16)
    oo = jnp.arange(11)
    e1 = ((jj[None, :, None] - oo[None, None, :]) ==
          jnp.arange(4)[:, None, None]).astype(jnp.float32)
    w = jnp.einsum('scapbq,bjo->apqjsoc', t, e1)
    return w.reshape(256, 704).astype(jnp.float32)


def _prep_w2(w2_mu, w2_sigma):
    z = jnp.zeros_like(w2_mu)
    w2 = jnp.concatenate([
        jnp.concatenate([w2_mu, z], axis=1),
        jnp.concatenate([z, w2_sigma], axis=1),
    ], axis=0)
    shifted = []
    for d1 in range(2):
        for d2 in range(2):
            shifted.append(
                jnp.pad(w2, ((0, 0), (0, 0), (d1, 1 - d1), (d2, 1 - d2))))
    wp = jnp.stack(shifted, 0)
    kk = jnp.arange(6)
    oo = jnp.arange(11)
    zz = jnp.arange(3)
    e2 = ((oo[None, :, None] - 2 * zz[None, None, :]) ==
          kk[:, None, None]).astype(jnp.float32)
    w = jnp.einsum('stcyk,koz->yocszt', wp, e2)
    return w.reshape(1056, 240).astype(jnp.float32)


def kernel(x, eps, w1_mu, b1_mu, w2_mu, b2_mu,
           w1_sigma, b1_sigma, w2_sigma, b2_sigma):
    x = x.reshape(-1, 28, 28).astype(jnp.float32)
    B = x.shape[0]
    blk = B if B <= _BLK else _BLK
    bpad = -(-B // blk) * blk

    xp = x.reshape(B, 14, 2, 14, 2).transpose(0, 1, 2, 4, 3)
    xp = jnp.pad(xp, ((0, bpad - B), (0, 2), (0, 0), (0, 0), (0, 2)))
    xp = xp.reshape(bpad, 16, 64)

    w1_all = _prep_w1(w1_mu, w1_sigma)
    b1 = jnp.concatenate([b1_mu, b1_sigma]).astype(jnp.float32)
    b1t = jnp.tile(b1, (11,)).reshape(1, 176)
    w2_all = _prep_w2(w2_mu, w2_sigma)
    b2 = jnp.concatenate([b2_mu, b2_sigma]).astype(jnp.float32)
    b2t = jnp.tile(b2, (3,)).reshape(1, 60)

    eps_r = jnp.transpose(eps.astype(jnp.float32), (0, 2, 3, 1))
    eps_r = jnp.pad(eps_r, ((0, bpad - B), (0, 0), (0, 0), (0, 0)))

    nsteps = bpad // blk
    z4, lp4 = pl.pallas_call(
        _fused_kernel,
        out_shape=(jax.ShapeDtypeStruct((bpad, 3, 3, 10), jnp.float32),
                   jax.ShapeDtypeStruct((bpad, 1, 1, 10), jnp.float32)),
        grid=(nsteps,),
        in_specs=[
            pl.BlockSpec((blk, 16, 64), lambda i: (i, 0, 0)),
            pl.BlockSpec((256, 704), lambda i: (0, 0)),
            pl.BlockSpec((1, 176), lambda i: (0, 0)),
            pl.BlockSpec((1056, 240), lambda i: (0, 0)),
            pl.BlockSpec((1, 60), lambda i: (0, 0)),
            pl.BlockSpec((blk, 3, 3, 10), lambda i: (i, 0, 0, 0)),
        ],
        out_specs=(pl.BlockSpec((blk, 3, 3, 10), lambda i: (i, 0, 0, 0)),
                   pl.BlockSpec((blk, 1, 1, 10), lambda i: (i, 0, 0, 0))),
        compiler_params=pltpu.CompilerParams(
            dimension_semantics=("parallel",),
            vmem_limit_bytes=_VMEM_LIMIT),
    )(xp, w1_all, b1t, w2_all, b2t, eps_r)

    z = jnp.transpose(z4[:B], (0, 3, 1, 2))
    score = jnp.sum(lp4[:B, 0, 0, :], axis=-1)
    return z, score
